# baseline (speedup 1.0000x reference)
import jax
import jax.numpy as jnp
from jax.experimental import pallas as pl


def _gat_body(x_ref, maskT_ref, w_ref, a1t_ref, a2t_ref, o_ref):
    x = x_ref[0]
    h = jnp.dot(x, w_ref[...], preferred_element_type=jnp.float32)
    h2 = jnp.dot(maskT_ref[...], h, preferred_element_type=jnp.float32)
    score = (
        jnp.sum(h * a1t_ref[...], axis=1) + jnp.sum(h2 * a2t_ref[...], axis=1)
    )
    hsum = jnp.sum(h, axis=0)
    o_ref[0] = jnp.maximum(score[:, None] * hsum[None, :], 0.0)


def kernel(inp, adj, W, a):
    b, t, n, fin = inp.shape
    fout = W.shape[1]
    bt = b * t
    x = inp.reshape(bt, n, fin)
    maskT = (adj > 0).astype(jnp.float32).T
    a1t = a[:fout, :].T
    a2t = a[fout:, :].T

    out = pl.pallas_call(
        _gat_body,
        grid=(bt,),
        in_specs=[
            pl.BlockSpec((1, n, fin), lambda i: (i, 0, 0)),
            pl.BlockSpec((n, n), lambda i: (0, 0)),
            pl.BlockSpec((fin, fout), lambda i: (0, 0)),
            pl.BlockSpec((n, fout), lambda i: (0, 0)),
            pl.BlockSpec((n, fout), lambda i: (0, 0)),
        ],
        out_specs=pl.BlockSpec((1, n, fout), lambda i: (i, 0, 0)),
        out_shape=jax.ShapeDtypeStruct((bt, n, fout), jnp.float32),
    )(x, maskT, W, a1t, a2t)
    return out.reshape(b, t, n, fout)


# speedup vs baseline: 1.0087x; 1.0087x over previous
import jax
import jax.numpy as jnp
from jax.experimental import pallas as pl
from jax.experimental.pallas import tpu as pltpu


def _gat_body(x_ref, maskT_ref, w_ref, a1t_ref, a2t_ref, o_ref):
    x = x_ref[0]
    h = jnp.dot(x, w_ref[...], preferred_element_type=jnp.float32)
    h2 = jnp.dot(
        maskT_ref[...], h.astype(jnp.bfloat16),
        preferred_element_type=jnp.float32,
    )
    score = (
        jnp.sum(h * a1t_ref[...], axis=1) + jnp.sum(h2 * a2t_ref[...], axis=1)
    )
    hsum = jnp.sum(h, axis=0)
    o_ref[0] = jnp.maximum(score[:, None] * hsum[None, :], 0.0)


def kernel(inp, adj, W, a):
    b, t, n, fin = inp.shape
    fout = W.shape[1]
    bt = b * t
    x = inp.reshape(bt, n, fin)
    maskT = (adj > 0).astype(jnp.bfloat16).T
    a1t = a[:fout, :].T
    a2t = a[fout:, :].T

    out = pl.pallas_call(
        _gat_body,
        grid=(bt,),
        in_specs=[
            pl.BlockSpec((1, n, fin), lambda i: (i, 0, 0)),
            pl.BlockSpec((n, n), lambda i: (0, 0)),
            pl.BlockSpec((fin, fout), lambda i: (0, 0)),
            pl.BlockSpec((n, fout), lambda i: (0, 0)),
            pl.BlockSpec((n, fout), lambda i: (0, 0)),
        ],
        out_specs=pl.BlockSpec((1, n, fout), lambda i: (i, 0, 0)),
        out_shape=jax.ShapeDtypeStruct((bt, n, fout), jnp.float32),
        compiler_params=pltpu.CompilerParams(
            dimension_semantics=(pltpu.PARALLEL,),
        ),
    )(x, maskT, W, a1t, a2t)
    return out.reshape(b, t, n, fout)


# speedup vs baseline: 1.3192x; 1.3078x over previous
import jax
import jax.numpy as jnp
from jax.experimental import pallas as pl
from jax.experimental.pallas import tpu as pltpu

_G = 8


def _gat_body(x_ref, maskT_ref, w_ref, a1t_ref, a2t_ref, o_ref, hc_ref):
    g, n, fin = x_ref.shape
    fout = w_ref.shape[1]
    x = x_ref[...].reshape(g * n, fin)
    h = jnp.dot(x, w_ref[...], preferred_element_type=jnp.float32)
    h3 = h.reshape(g, n, fout)
    for k in range(g):
        hc_ref[:, k * fout:(k + 1) * fout] = h3[k].astype(jnp.bfloat16)
    h2c = jnp.dot(maskT_ref[...], hc_ref[...],
                  preferred_element_type=jnp.float32)
    a1t = a1t_ref[...]
    a2t = a2t_ref[...]
    for k in range(g):
        hk = h3[k]
        score = (jnp.sum(hk * a1t, axis=1)
                 + jnp.sum(h2c[:, k * fout:(k + 1) * fout] * a2t, axis=1))
        hsum = jnp.sum(hk, axis=0)
        o_ref[k] = jnp.maximum(score[:, None] * hsum[None, :], 0.0)


def kernel(inp, adj, W, a):
    b, t, n, fin = inp.shape
    fout = W.shape[1]
    bt = b * t
    x = inp.reshape(bt, n, fin)
    maskT = (adj > 0).astype(jnp.bfloat16).T
    a1t = a[:fout, :].T
    a2t = a[fout:, :].T

    out = pl.pallas_call(
        _gat_body,
        grid=(bt // _G,),
        in_specs=[
            pl.BlockSpec((_G, n, fin), lambda i: (i, 0, 0)),
            pl.BlockSpec((n, n), lambda i: (0, 0)),
            pl.BlockSpec((fin, fout), lambda i: (0, 0)),
            pl.BlockSpec((n, fout), lambda i: (0, 0)),
            pl.BlockSpec((n, fout), lambda i: (0, 0)),
        ],
        out_specs=pl.BlockSpec((_G, n, fout), lambda i: (i, 0, 0)),
        out_shape=jax.ShapeDtypeStruct((bt, n, fout), jnp.float32),
        scratch_shapes=[pltpu.VMEM((n, _G * fout), jnp.bfloat16)],
        compiler_params=pltpu.CompilerParams(
            dimension_semantics=(pltpu.PARALLEL,),
        ),
    )(x, maskT, W, a1t, a2t)
    return out.reshape(b, t, n, fout)


# speedup vs baseline: 1.9448x; 1.4742x over previous
import jax
import jax.numpy as jnp
from jax.experimental import pallas as pl
from jax.experimental.pallas import tpu as pltpu


def _gat_body(x_ref, adj_ref, w_ref, a_ref, o_ref, hc_ref, hcb_ref):
    _, g, n, fin = x_ref.shape
    fout = w_ref.shape[1]
    x = x_ref[0].reshape(g * n, fin)
    h = jnp.dot(x, w_ref[...], preferred_element_type=jnp.float32)
    h3 = h.reshape(g, n, fout)
    for k in range(g):
        hc_ref[:, k * fout:(k + 1) * fout] = h3[k]
        hcb_ref[:, k * fout:(k + 1) * fout] = h3[k].astype(jnp.bfloat16)
    mask = (adj_ref[...] > 0).astype(jnp.bfloat16)
    h2c = jax.lax.dot_general(
        mask, hcb_ref[...], (((0,), (0,)), ((), ())),
        preferred_element_type=jnp.float32,
    )
    a1t = a_ref[:fout, :].T
    a2t = a_ref[fout:, :].T
    a1rep = jnp.concatenate([a1t] * g, axis=1)
    a2rep = jnp.concatenate([a2t] * g, axis=1)
    hcf = hc_ref[...]
    p = hcf * a1rep + h2c * a2rep
    rows = jax.lax.broadcasted_iota(jnp.int32, (g * fout, g), 0)
    cols = jax.lax.broadcasted_iota(jnp.int32, (g * fout, g), 1)
    bsel = (rows // fout == cols).astype(jnp.float32)
    scores = jnp.dot(p, bsel, preferred_element_type=jnp.float32)
    hsums = jnp.sum(hcf, axis=0, keepdims=True)
    for k in range(g):
        sc = jnp.broadcast_to(scores[:, k:k + 1], (n, fout))
        hs = jnp.broadcast_to(hsums[:, k * fout:(k + 1) * fout], (n, fout))
        o_ref[0, k] = jnp.maximum(sc * hs, 0.0)


def kernel(inp, adj, W, a):
    b, t, n, fin = inp.shape
    fout = W.shape[1]

    return pl.pallas_call(
        _gat_body,
        grid=(b,),
        in_specs=[
            pl.BlockSpec((1, t, n, fin), lambda i: (i, 0, 0, 0)),
            pl.BlockSpec((n, n), lambda i: (0, 0)),
            pl.BlockSpec((fin, fout), lambda i: (0, 0)),
            pl.BlockSpec((2 * fout, n), lambda i: (0, 0)),
        ],
        out_specs=pl.BlockSpec((1, t, n, fout), lambda i: (i, 0, 0, 0)),
        out_shape=jax.ShapeDtypeStruct((b, t, n, fout), jnp.float32),
        scratch_shapes=[
            pltpu.VMEM((n, t * fout), jnp.float32),
            pltpu.VMEM((n, t * fout), jnp.bfloat16),
        ],
        compiler_params=pltpu.CompilerParams(
            dimension_semantics=(pltpu.PARALLEL,),
        ),
    )(inp, adj, W, a)
